# baseline (speedup 1.0000x reference)
import functools

import jax
import jax.numpy as jnp
from jax import lax
from jax.experimental import pallas as pl
from jax.experimental.pallas import tpu as pltpu
from jax.experimental.pallas import tpu_sc as plsc

N_NODES = 10000
N_EDGES = 160000
DIM = 256
N_LAYER = 3

NP = 10240
NB = 4
BN = NP // NB
RPT = BN // 16
MT = 10640
M2 = 16 * MT
CH = 128
RBLK = 3
CAPB = RBLK * 8 * CH
NCHD = 80
DCH = 128


def _mesh():
    return plsc.VectorSubcoreMesh(core_axis_name="c", subcore_axis_name="s")


def _sc_deg(dst_tiles, zdeg, ones_blk):
    @functools.partial(
        pl.kernel, mesh=_mesh(),
        out_type=jax.ShapeDtypeStruct((2, NP, 16), jnp.float32),
        scratch_types=[
            pltpu.VMEM_SHARED((NP, 16), jnp.float32),
            pltpu.VMEM((NCHD, DCH), jnp.int32),
            pltpu.VMEM((DCH, 16), jnp.float32),
        ],
    )
    def deg_k(dst_hbm, zdeg_hbm, ones_hbm, degp_hbm, deg_sh, rowv, onesv):
        c = lax.axis_index("c")
        s = lax.axis_index("s")
        base = s * 640
        pltpu.sync_copy(zdeg_hbm, deg_sh.at[pl.ds(base, 640)])
        pltpu.sync_copy(dst_hbm.at[s], rowv)
        pltpu.sync_copy(ones_hbm, onesv)
        plsc.subcore_barrier()

        def dchunk(j, _):
            pltpu.sync_copy(onesv, deg_sh.at[rowv.at[j]], add=True)
            return 0
        lax.fori_loop(c * (NCHD // 2), (c + 1) * (NCHD // 2), dchunk, 0)
        plsc.subcore_barrier()
        pltpu.sync_copy(deg_sh.at[pl.ds(base, 640)],
                        degp_hbm.at[c, pl.ds(base, 640)])

    return deg_k(dst_tiles, zdeg, ones_blk)


def _sc_partition(msg_in):
    @functools.partial(
        pl.kernel, mesh=_mesh(),
        compiler_params=pltpu.CompilerParams(needs_layout_passes=False),
        out_type=jax.ShapeDtypeStruct((NB, 16, 2, RBLK, 8, CH),
                                      jnp.int32),
        scratch_types=[
            pltpu.VMEM((2, MT), jnp.int32),
            pltpu.VMEM((2, RBLK * 8, CH), jnp.int32),
            pltpu.VMEM((2 * CAPB,), jnp.int32),
        ],
    )
    def part_k(msg_hbm, mlist_hbm, msgv, mbuf, pbuf):
        c = lax.axis_index("c")
        s = lax.axis_index("s")
        pltpu.sync_copy(msg_hbm.at[s], msgv)

        lanes = jnp.arange(16, dtype=jnp.int32)
        zv = jnp.zeros((16,), jnp.int32)
        ov = jnp.ones((16,), jnp.int32)

        gdn = lax.GatherDimensionNumbers(offset_dims=(),
                                         collapsed_slice_dims=(0,),
                                         start_index_map=(0,))

        def gather16(x, idx):
            return lax.gather(x, idx[:, None], gdn, (1,),
                              mode=lax.GatherScatterMode.PROMISE_IN_BOUNDS)

        def cumsum16(x):
            for sh in (1, 2, 4, 8):
                idx = jnp.maximum(lanes - sh, 0)
                x = x + jnp.where(lanes >= sh, gather16(x, idx), zv)
            return x

        last = jnp.full((16,), 15, jnp.int32)

        def compact(i, curs):
            b = i * 16
            col = msgv[0, pl.ds(b, 16)]
            dst = msgv[1, pl.ds(b, 16)]
            qv = dst // BN
            packed = (col << 12) | ((dst - qv * BN) & 4095)
            out = []
            for qq in range(2):
                inb = qv == c * 2 + qq
                mi = jnp.where(inb, ov, zv)
                cum = cumsum16(mi)
                pos = curs[qq] + cum - mi
                plsc.store_scatter(pbuf, [qq * CAPB + pos], packed,
                                   mask=inb & (pos < CAPB))
                out.append(curs[qq] + gather16(cum, last))
            return tuple(out)
        curs = lax.fori_loop(0, MT // 16, compact, (zv, zv))

        noop = jnp.full((16,), N_NODES << 12, jnp.int32)
        for qq in range(2):
            def fill(i, _):
                pos = i * 16 + lanes
                plsc.store_scatter(pbuf, [qq * CAPB + pos], noop,
                                   mask=pos >= curs[qq])
                return 0
            lax.fori_loop(0, CAPB // 16, fill, 0)

        for qq in range(2):
            q = c * 2 + qq

            def unpack(i, _):
                p0 = i * 16
                v = pbuf[pl.ds(qq * CAPB + p0, 16)]
                mbuf[0, p0 >> 7, pl.ds(p0 & 127, 16)] = v >> 12
                mbuf[1, p0 >> 7, pl.ds(p0 & 127, 16)] = v & 4095
                return 0
            lax.fori_loop(0, CAPB // 16, unpack, 0)
            for blk in range(RBLK):
                rows = pl.ds(blk * 8, 8)
                pltpu.sync_copy(mbuf.at[0, rows],
                                mlist_hbm.at[q, s, 0, blk])
                pltpu.sync_copy(mbuf.at[1, rows],
                                mlist_hbm.at[q, s, 1, blk])

    return part_k(msg_in)


def _sc_layer(mlist, y_in, zacc):
    @functools.partial(
        pl.kernel, mesh=_mesh(),
        out_type=jax.ShapeDtypeStruct((NP, 2, 128), jnp.float32),
        scratch_types=[
            pltpu.VMEM_SHARED((BN, 2, 128), jnp.float32),
            pltpu.VMEM((CH, 2, 128), jnp.float32),
            pltpu.VMEM((CH, 2, 128), jnp.float32),
            pltpu.VMEM((8, CH), jnp.int32),
            pltpu.VMEM((8, CH), jnp.int32),
            pltpu.SemaphoreType.DMA,
            pltpu.SemaphoreType.DMA,
            pltpu.SemaphoreType.DMA,
            pltpu.SemaphoreType.DMA,
        ],
    )
    def layer_k(ml_hbm, y_hbm, z_hbm, out_hbm,
          acc_sh, gbufa, gbufb, ivc, ivr,
          gsa, gsb, ssa, ssb):
        c = lax.axis_index("c")
        s = lax.axis_index("s")
        base = s * RPT
        bufs = (gbufa, gbufb)
        gsem = (gsa, gsb)
        ssem = (ssa, ssb)

        for qq in range(2):
            q = c * 2 + qq
            pltpu.sync_copy(z_hbm, acc_sh.at[pl.ds(base, RPT)])
            plsc.subcore_barrier()

            for blk in range(RBLK):
                pltpu.sync_copy(ml_hbm.at[q, s, 0, blk], ivc)
                pltpu.sync_copy(ml_hbm.at[q, s, 1, blk], ivr)
                hg = [None] * 8
                hs = [None] * 8
                hg[0] = pltpu.async_copy(y_hbm.at[ivc.at[0]], gbufa, gsa)
                for j in range(8):
                    b = j & 1
                    if j + 1 < 8:
                        if j >= 1:
                            hs[j - 1].wait()
                        hg[j + 1] = pltpu.async_copy(
                            y_hbm.at[ivc.at[j + 1]], bufs[1 - b],
                            gsem[1 - b])
                    hg[j].wait()
                    hs[j] = pltpu.async_copy(
                        bufs[b], acc_sh.at[ivr.at[j]], ssem[b],
                        add=True)
                hs[6].wait()
                hs[7].wait()
            plsc.subcore_barrier()
            pltpu.sync_copy(acc_sh.at[pl.ds(base, RPT)],
                            out_hbm.at[pl.ds(q * BN + base, RPT)])
            plsc.subcore_barrier()

    return layer_k(mlist, y_in, zacc)


def _deg_math_body(dz_ref, degw_ref):
    deg = dz_ref[:, 0, 0:1]
    degw_ref[...] = jnp.broadcast_to(deg, (NP, 128))


def _tc_deg_math(degz):
    return pl.pallas_call(
        _deg_math_body,
        out_shape=jax.ShapeDtypeStruct((NP, 128), jnp.float32),
    )(degz)


def _mm_body(f_ref, l_ref, dw_ref, y_ref):
    x0 = jnp.dot(f_ref[...], l_ref[...], preferred_element_type=jnp.float32)
    y_ref[...] = lax.rsqrt(dw_ref[...][:, :1]) * x0


def _tc_matmul_scale(fpad, lin, degw):
    blk = 640
    return pl.pallas_call(
        _mm_body,
        grid=(NP // blk,),
        in_specs=[
            pl.BlockSpec((blk, DIM), lambda i: (i, 0)),
            pl.BlockSpec((DIM, DIM), lambda i: (0, 0)),
            pl.BlockSpec((blk, 128), lambda i: (i, 0)),
        ],
        out_specs=pl.BlockSpec((blk, DIM), lambda i: (i, 0)),
        out_shape=jax.ShapeDtypeStruct((NP, DIM), jnp.float32),
    )(fpad, lin, degw)


def _scale_body(z_ref, dw_ref, y_ref):
    y_ref[...] = z_ref[...] / dw_ref[...][:, :1]


def _tc_interlayer(z, degw):
    blk = 640
    return pl.pallas_call(
        _scale_body,
        grid=(NP // blk,),
        in_specs=[
            pl.BlockSpec((blk, DIM), lambda i: (i, 0)),
            pl.BlockSpec((blk, 128), lambda i: (i, 0)),
        ],
        out_specs=pl.BlockSpec((blk, DIM), lambda i: (i, 0)),
        out_shape=jax.ShapeDtypeStruct((NP, DIM), jnp.float32),
    )(z, degw)


def _final_body(z1_ref, z2_ref, z3_ref, dw_ref, lp_ref, x1_ref, x2_ref,
                x3_ref):
    di = lax.rsqrt(dw_ref[...][:, :1])
    x1_ref[...] = di * z1_ref[...]
    x2_ref[...] = di * z2_ref[...]
    x3 = di * z3_ref[...]
    x3_ref[...] = x3
    m = jnp.max(x3, axis=1, keepdims=True)
    lse = m + jnp.log(jnp.sum(jnp.exp(x3 - m), axis=1, keepdims=True))
    lp_ref[...] = x3 - lse


def _tc_final(z1, z2, z3, degw10k):
    blk = 1000
    out = jax.ShapeDtypeStruct((N_NODES, DIM), jnp.float32)
    return pl.pallas_call(
        _final_body,
        grid=(N_NODES // blk,),
        in_specs=[pl.BlockSpec((blk, DIM), lambda i: (i, 0))] * 3
        + [pl.BlockSpec((blk, 128), lambda i: (i, 0))],
        out_specs=[pl.BlockSpec((blk, DIM), lambda i: (i, 0))] * 4,
        out_shape=(out, out, out, out),
    )(z1, z2, z3, degw10k)


def kernel(features, adj, lin):
    src = adj[0].astype(jnp.int32)
    dst = adj[1].astype(jnp.int32)
    loops = jnp.arange(N_NODES, dtype=jnp.int32)
    padm = jnp.full((M2 - N_EDGES - N_NODES,), N_NODES, jnp.int32)
    col_all = jnp.concatenate([src, loops, padm])
    dst_all = jnp.concatenate([dst, loops, padm])
    msg_in = (jnp.stack([col_all, dst_all])
              .reshape(2, 16, MT).transpose(1, 0, 2))

    zacc = jnp.zeros((RPT, 2, 128), jnp.float32)

    mlist = _sc_partition(msg_in)
    ones_tab = (jnp.zeros((NP, DIM), jnp.float32).at[:N_NODES].set(1.0)
                .reshape(NP, 2, 128))
    degz = _sc_layer(mlist, ones_tab, zacc)
    degw = _tc_deg_math(degz)

    fpad = jnp.zeros((NP, DIM), jnp.float32).at[:N_NODES].set(features)
    y0 = _tc_matmul_scale(fpad, lin, degw)

    def layer(y):
        return _sc_layer(mlist, y.reshape(NP, 2, 128),
                         zacc).reshape(NP, DIM)

    z1 = layer(y0)
    y1 = _tc_interlayer(z1, degw)
    z2 = layer(y1)
    y2 = _tc_interlayer(z2, degw)
    z3 = layer(y2)

    lp, x1, x2, x3 = _tc_final(z1[:N_NODES], z2[:N_NODES], z3[:N_NODES],
                               degw[:N_NODES])
    return (lp, x3, x1, x2, x3)


# speedup vs baseline: 2.2611x; 2.2611x over previous
import functools

import jax
import jax.numpy as jnp
from jax import lax
from jax.experimental import pallas as pl
from jax.experimental.pallas import tpu as pltpu
from jax.experimental.pallas import tpu_sc as plsc

N_NODES = 10000
N_EDGES = 160000
DIM = 256
HALF = 128
N_LAYER = 3

NP = 10240
RT = 640
RTW = 32
N_MSG = N_EDGES + N_NODES
CHUNK = 128
NCH = 88
STG = 8
M_PAD = 16 * NCH * CHUNK


def _sc_deg(row_idx, zdeg):
    mesh = plsc.VectorSubcoreMesh(core_axis_name="c", subcore_axis_name="s")

    @functools.partial(
        pl.kernel, mesh=mesh,
        out_type=jax.ShapeDtypeStruct((2, NP, 16), jnp.float32),
        scratch_types=[
            pltpu.VMEM_SHARED((NP, 16), jnp.float32),
            pltpu.VMEM((NCH, CHUNK), jnp.int32),
            pltpu.VMEM((CHUNK, 16), jnp.float32),
        ],
    )
    def k(row_hbm, zdeg_hbm, degp_hbm, deg_sh, rowv, onesv):
        c = lax.axis_index("c")
        s = lax.axis_index("s")
        base = s * RT
        pltpu.sync_copy(zdeg_hbm, deg_sh.at[pl.ds(base, RT)])
        pltpu.sync_copy(row_hbm.at[s], rowv)

        def fill(r, _):
            onesv[r, :] = jnp.full((16,), 1.0, jnp.float32)
            return 0
        lax.fori_loop(0, CHUNK, fill, 0)
        plsc.subcore_barrier()

        def body(j, _):
            pltpu.sync_copy(onesv, deg_sh.at[rowv.at[j]], add=True)
            return 0
        lax.fori_loop(c * (NCH // 2), (c + 1) * (NCH // 2), body, 0)
        plsc.subcore_barrier()
        pltpu.sync_copy(deg_sh.at[pl.ds(base, RT)],
                        degp_hbm.at[c, pl.ds(base, RT)])

    return k(row_idx, zdeg)


def _sc_layers(col_idx, row_idx, y0_tab, dinv2w, zrow):
    mesh = plsc.VectorSubcoreMesh(core_axis_name="c", subcore_axis_name="s")
    ytab = jax.ShapeDtypeStruct((2 * NP, HALF), jnp.float32)

    @functools.partial(
        pl.kernel, mesh=mesh,
        out_type=(ytab, ytab, ytab),
        scratch_types=[
            pltpu.VMEM_SHARED((NP, HALF), jnp.float32),
            pltpu.VMEM((STG, CHUNK), jnp.int32),
            pltpu.VMEM((STG, CHUNK), jnp.int32),
            pltpu.VMEM((CHUNK, HALF), jnp.float32),
            pltpu.VMEM((CHUNK, HALF), jnp.float32),
            pltpu.VMEM((RTW, HALF), jnp.float32),
            pltpu.VMEM((RTW, 16), jnp.float32),
            pltpu.SemaphoreType.DMA,
            pltpu.SemaphoreType.DMA,
            pltpu.SemaphoreType.DMA,
            pltpu.SemaphoreType.DMA,
        ],
    )
    def k(col_hbm, row_hbm, y0_hbm, d2_hbm, z_hbm,
          y1_hbm, y2_hbm, y3_hbm,
          acc_sh, colv, rowv, gbufa, gbufb, wacc, d2v,
          gsa, gsb, ssa, ssb):
        c = lax.axis_index("c")
        s = lax.axis_index("s")
        base = s * RT

        for y_in, y_out in ((y0_hbm, y1_hbm), (y1_hbm, y2_hbm),
                            (y2_hbm, y3_hbm)):
            pltpu.sync_copy(z_hbm, acc_sh.at[pl.ds(base, RT)])
            plsc.subcore_barrier()

            def stage(st, _):
                pltpu.sync_copy(col_hbm.at[c, s, pl.ds(st * STG, STG)], colv)
                pltpu.sync_copy(row_hbm.at[s, pl.ds(st * STG, STG)], rowv)
                bufs = (gbufa, gbufb)
                gsem = (gsa, gsb)
                ssem = (ssa, ssb)
                hg = [None] * STG
                hs = [None] * STG
                hg[0] = pltpu.async_copy(y_in.at[colv.at[0]], gbufa, gsa)
                for j in range(STG):
                    b = j & 1
                    if j + 1 < STG:
                        if j >= 1:
                            hs[j - 1].wait()
                        hg[j + 1] = pltpu.async_copy(
                            y_in.at[colv.at[j + 1]], bufs[1 - b], gsem[1 - b])
                    hg[j].wait()
                    hs[j] = pltpu.async_copy(
                        bufs[b], acc_sh.at[rowv.at[j]], ssem[b], add=True)
                hs[STG - 2].wait()
                hs[STG - 1].wait()
                return 0
            lax.fori_loop(0, NCH // STG, stage, 0)
            plsc.subcore_barrier()

            def wb(blk, _):
                off = base + blk * RTW
                pltpu.sync_copy(acc_sh.at[pl.ds(off, RTW)], wacc)
                pltpu.sync_copy(d2_hbm.at[pl.ds(off, RTW)], d2v)

                def scale(r, _):
                    for k8 in range(HALF // 16):
                        sl = pl.ds(k8 * 16, 16)
                        wacc[r, sl] = wacc[r, sl] * d2v[r, :]
                    return 0
                lax.fori_loop(0, RTW, scale, 0)
                pltpu.sync_copy(wacc, y_out.at[pl.ds(c * NP + off, RTW)])
                return 0
            lax.fori_loop(0, RT // RTW, wb, 0)
            plsc.subcore_barrier()

    return k(col_idx, row_idx, y0_tab, dinv2w, zrow)


def _deg_math_body(dp_ref, d2_ref, degw_ref):
    deg = dp_ref[0] + dp_ref[1]
    d2_ref[...] = jnp.where(deg > 0, 1.0 / deg, 0.0)
    degw_ref[...] = jnp.broadcast_to(deg[:, :1], (NP, HALF))


def _tc_deg_math(deg_part):
    return pl.pallas_call(
        _deg_math_body,
        out_shape=(jax.ShapeDtypeStruct((NP, 16), jnp.float32),
                   jax.ShapeDtypeStruct((NP, HALF), jnp.float32)),
    )(deg_part)


def _mm_body(f_ref, l_ref, dw_ref, y_ref):
    x0 = jnp.dot(f_ref[...], l_ref[...], preferred_element_type=jnp.float32)
    y_ref[...] = lax.rsqrt(dw_ref[...][:, :1]) * x0


def _tc_matmul_scale(features, lin, degw10k):
    blk = 1000
    return pl.pallas_call(
        _mm_body,
        grid=(N_NODES // blk,),
        in_specs=[
            pl.BlockSpec((blk, DIM), lambda i: (i, 0)),
            pl.BlockSpec((DIM, DIM), lambda i: (0, 0)),
            pl.BlockSpec((blk, HALF), lambda i: (i, 0)),
        ],
        out_specs=pl.BlockSpec((blk, DIM), lambda i: (i, 0)),
        out_shape=jax.ShapeDtypeStruct((N_NODES, DIM), jnp.float32),
    )(features, lin, degw10k)


def _final_body(y1_ref, y2_ref, y3_ref, dw_ref, lp_ref, x1_ref, x2_ref,
                x3_ref):
    sd = jnp.sqrt(dw_ref[...][:, :1])
    x1_ref[...] = sd * y1_ref[...]
    x2_ref[...] = sd * y2_ref[...]
    x3 = sd * y3_ref[...]
    x3_ref[...] = x3
    m = jnp.max(x3, axis=1, keepdims=True)
    lse = m + jnp.log(jnp.sum(jnp.exp(x3 - m), axis=1, keepdims=True))
    lp_ref[...] = x3 - lse


def _tc_final(y1, y2, y3, degw10k):
    blk = 1000
    out = jax.ShapeDtypeStruct((N_NODES, DIM), jnp.float32)
    return pl.pallas_call(
        _final_body,
        grid=(N_NODES // blk,),
        in_specs=[pl.BlockSpec((blk, DIM), lambda i: (i, 0))] * 3
        + [pl.BlockSpec((blk, HALF), lambda i: (i, 0))],
        out_specs=[pl.BlockSpec((blk, DIM), lambda i: (i, 0))] * 4,
        out_shape=(out, out, out, out),
    )(y1, y2, y3, degw10k)


def _untab(yt):
    return (yt.reshape(2, NP, HALF)[:, :N_NODES, :]
            .transpose(1, 0, 2).reshape(N_NODES, DIM))


def kernel(features, adj, lin):
    src = adj[0].astype(jnp.int32)
    dst = adj[1].astype(jnp.int32)
    loops = jnp.arange(N_NODES, dtype=jnp.int32)
    pad = jnp.full((M_PAD - N_MSG,), N_NODES, jnp.int32)
    row = jnp.concatenate([dst, loops, pad])
    col = jnp.concatenate([src, loops, pad])
    row_idx = row.reshape(16, NCH, CHUNK)
    col_idx = jnp.stack([col, col + NP]).reshape(2, 16, NCH, CHUNK)

    zdeg = jnp.zeros((RT, 16), jnp.float32)
    zrow = jnp.zeros((RT, HALF), jnp.float32)

    deg_part = _sc_deg(row_idx, zdeg)
    dinv2w, degw = _tc_deg_math(deg_part)
    degw10k = degw[:N_NODES]

    y0 = _tc_matmul_scale(features, lin, degw10k)
    y0_tab = (jnp.zeros((2, NP, HALF), jnp.float32)
              .at[:, :N_NODES, :]
              .set(y0.reshape(N_NODES, 2, HALF).transpose(1, 0, 2))
              .reshape(2 * NP, HALF))

    y1t, y2t, y3t = _sc_layers(col_idx, row_idx, y0_tab, dinv2w, zrow)

    lp, x1, x2, x3 = _tc_final(_untab(y1t), _untab(y2t), _untab(y3t), degw10k)
    return (lp, x3, x1, x2, x3)
